# baseline (speedup 1.0000x reference)
import functools
import jax
import jax.numpy as jnp
from jax.experimental import pallas as pl

K = 20
B = 8
NPTS = 2048
R = 256
BIG = 3.0e38


def _edge_conv_body(x_full_ref, x_row_ref, Wd_ref, Wn_ref, ba_ref, Wb_ref,
                    bb_ref, out_ref):
    xall = x_full_ref[0]
    xr = x_row_ref[0]
    f32 = jnp.float32
    PREC = jax.lax.Precision.HIGHEST

    sqall = jnp.sum(xall * xall, axis=1)
    sqr = jnp.sum(xr * xr, axis=1)
    dot = jax.lax.dot_general(xr, xall, (((1,), (1,)), ((), ())),
                              preferred_element_type=f32, precision=PREC)
    d = sqr[:, None] + sqall[None, :] - 2.0 * dot

    a = jax.lax.dot_general(xr, Wd_ref[...], (((1,), (0,)), ((), ())),
                            preferred_element_type=f32, precision=PREC)
    a = a + ba_ref[...]
    y = jax.lax.dot_general(xall, Wn_ref[...], (((1,), (0,)), ((), ())),
                            preferred_element_type=f32, precision=PREC)
    y_hi = y.astype(jnp.bfloat16)
    y_lo = (y - y_hi.astype(f32)).astype(jnp.bfloat16)

    Wb = Wb_ref[...]
    iota = jax.lax.broadcasted_iota(jnp.int32, (R, NPTS), 1)
    HALF = NPTS // 2
    iotap = jax.lax.broadcasted_iota(jnp.int32, (R, HALF), 1)
    dlow, dhigh = d[:, :HALF], d[:, HALF:]
    lo = jnp.minimum(dlow, dhigh)
    hi = jnp.maximum(dlow, dhigh)
    low_first = dlow <= dhigh
    acc = jnp.full((R, Wb.shape[1]), -BIG, dtype=f32)
    for _ in range(K):
        mj = jnp.min(lo, axis=1)
        cand = jnp.where(lo <= mj[:, None], iotap, HALF)
        j = jnp.min(cand, axis=1)
        ohp = iotap == j[:, None]
        sel_low = jnp.max(jnp.where(ohp & low_first, 1, 0), axis=1)
        idx = j + (1 - sel_low) * HALF
        lo = jnp.where(ohp, hi, lo)
        hi = jnp.where(ohp, BIG, hi)
        low_first = low_first ^ ohp
        oh16 = (iota == idx[:, None]).astype(jnp.bfloat16)
        g = (jax.lax.dot_general(oh16, y_hi, (((1,), (0,)), ((), ())),
                                 preferred_element_type=f32)
             + jax.lax.dot_general(oh16, y_lo, (((1,), (0,)), ((), ())),
                                   preferred_element_type=f32))
        h = jax.lax.dot_general(jnp.maximum(a + g, 0.0), Wb,
                                (((1,), (0,)), ((), ())),
                                preferred_element_type=f32, precision=PREC)
        acc = jnp.maximum(acc, h)
    out_ref[0] = acc + bb_ref[...]


def _edge_conv(x, Wd, Wn, ba, Wb, bb):
    C = x.shape[-1]
    H = Wd.shape[1]
    F = Wb.shape[1]
    grid = (B, NPTS // R)
    return pl.pallas_call(
        _edge_conv_body,
        grid=grid,
        in_specs=[
            pl.BlockSpec((1, NPTS, C), lambda b, r: (b, 0, 0)),
            pl.BlockSpec((1, R, C), lambda b, r: (b, r, 0)),
            pl.BlockSpec((C, H), lambda b, r: (0, 0)),
            pl.BlockSpec((C, H), lambda b, r: (0, 0)),
            pl.BlockSpec((1, H), lambda b, r: (0, 0)),
            pl.BlockSpec((H, F), lambda b, r: (0, 0)),
            pl.BlockSpec((1, F), lambda b, r: (0, 0)),
        ],
        out_specs=pl.BlockSpec((1, R, F), lambda b, r: (b, r, 0)),
        out_shape=jax.ShapeDtypeStruct((B, NPTS, F), jnp.float32),
    )(x, x, Wd, Wn, ba, Wb, bb)


def _maxpool_body(x1_ref, x2_ref, x3_ref, x4_ref, out_ref):
    m1 = jnp.max(x1_ref[0], axis=0)
    m2 = jnp.max(x2_ref[0], axis=0)
    m3 = jnp.max(x3_ref[0], axis=0)
    m4 = jnp.max(x4_ref[0], axis=0)
    out_ref[...] = jnp.concatenate([m1, m2, m3, m4], axis=0)[None, None, :]


def _maxpool(x1, x2, x3, x4):
    return pl.pallas_call(
        _maxpool_body,
        grid=(B,),
        in_specs=[
            pl.BlockSpec((1, NPTS, 64), lambda b: (b, 0, 0)),
            pl.BlockSpec((1, NPTS, 64), lambda b: (b, 0, 0)),
            pl.BlockSpec((1, NPTS, 64), lambda b: (b, 0, 0)),
            pl.BlockSpec((1, NPTS, 128), lambda b: (b, 0, 0)),
        ],
        out_specs=pl.BlockSpec((1, 1, 320), lambda b: (b, 0, 0)),
        out_shape=jax.ShapeDtypeStruct((B, 1, 320), jnp.float32),
    )(x1, x2, x3, x4).reshape(B, 320)


def _decoder_body(pooled_ref, tooth_ref, emb_table_ref, cembT_ref, cemb_b_ref,
                  encWa_ref, encWb_ref, enc_b_ref, dW1_ref, db1_ref,
                  dW2_ref, db2_ref, out_ref):
    f32 = jnp.float32
    PREC = jax.lax.Precision.HIGHEST

    def mm(u, v):
        return jax.lax.dot_general(u, v, (((1,), (0,)), ((), ())),
                                   preferred_element_type=f32, precision=PREC)

    tooth = tooth_ref[...]
    oh = (jax.lax.broadcasted_iota(jnp.int32, (B, 33), 1)
          == tooth).astype(f32)
    emb = mm(oh, emb_table_ref[...])
    emb = mm(emb, cembT_ref[...]) + cemb_b_ref[...]
    h = mm(pooled_ref[...], encWa_ref[...]) + mm(emb, encWb_ref[...])
    h = jnp.maximum(h + enc_b_ref[...], 0.0)
    h = jnp.maximum(mm(h, dW1_ref[...]) + db1_ref[...], 0.0)
    out_ref[...] = mm(h, dW2_ref[...]) + db2_ref[...]


def _decoder(pooled, tooth_n, emb_table, conv_emb_W, conv_emb_b,
             enc_W, enc_b, dec_W1, dec_b1, dec_W2, dec_b2):
    return pl.pallas_call(
        _decoder_body,
        out_shape=jax.ShapeDtypeStruct((B, 3072), jnp.float32),
    )(pooled, tooth_n.reshape(B, 1), emb_table, conv_emb_W.T,
      conv_emb_b.reshape(1, 64), enc_W[:320], enc_W[320:],
      enc_b.reshape(1, 512), dec_W1, dec_b1.reshape(1, 1024),
      dec_W2, dec_b2.reshape(1, 3072))


def kernel(pos, batch, tooth_n, emb_table, conv_emb_W, conv_emb_b,
           W1a, b1a, W1b, b1b, W2a, b2a, W2b, b2b,
           W3a, b3a, W3b, b3b, W4a, b4a, W4b, b4b,
           enc_W, enc_b, dec_W1, dec_b1, dec_W2, dec_b2):
    x0 = pos.reshape(B, NPTS, 3)
    x0 = jnp.concatenate([x0, jnp.zeros((B, NPTS, 5), jnp.float32)], axis=-1)

    def split(Wa, cpad=None):
        C = Wa.shape[0] // 2
        top, bot = Wa[:C], Wa[C:]
        Wd, Wn = top - bot, bot
        if cpad is not None and cpad > C:
            z = jnp.zeros((cpad - C, Wa.shape[1]), jnp.float32)
            Wd = jnp.concatenate([Wd, z], axis=0)
            Wn = jnp.concatenate([Wn, z], axis=0)
        return Wd, Wn

    Wd1, Wn1 = split(W1a, 8)
    x1 = _edge_conv(x0, Wd1, Wn1, b1a.reshape(1, -1), W1b, b1b.reshape(1, -1))
    Wd2, Wn2 = split(W2a)
    x2 = _edge_conv(x1, Wd2, Wn2, b2a.reshape(1, -1), W2b, b2b.reshape(1, -1))
    Wd3, Wn3 = split(W3a)
    x3 = _edge_conv(x2, Wd3, Wn3, b3a.reshape(1, -1), W3b, b3b.reshape(1, -1))
    Wd4, Wn4 = split(W4a)
    x4 = _edge_conv(x3, Wd4, Wn4, b4a.reshape(1, -1), W4b, b4b.reshape(1, -1))

    pooled = _maxpool(x1, x2, x3, x4)
    out = _decoder(pooled, tooth_n, emb_table, conv_emb_W, conv_emb_b,
                   enc_W, enc_b, dec_W1, dec_b1, dec_W2, dec_b2)
    return out.reshape(B, 1024, 3)


# speedup vs baseline: 1.1644x; 1.1644x over previous
import functools
import jax
import jax.numpy as jnp
from jax import lax
from jax.experimental import pallas as pl
from jax.experimental.pallas import tpu as pltpu
from jax.experimental.pallas import tpu_sc as plsc

K = 20
B = 8
NPTS = 2048
R = 256
BIG = 3.0e38
NIDX = B * K * NPTS


def _knn_body(x_full_ref, x_row_ref, Wd_ref, Wn_ref, ba_ref,
              idx_ref, y_ref, a_ref):
    xall = x_full_ref[0]
    xr = x_row_ref[0]
    f32 = jnp.float32
    PREC = jax.lax.Precision.HIGHEST
    boff = pl.program_id(0) * NPTS

    sqall = jnp.sum(xall * xall, axis=1)
    sqr = jnp.sum(xr * xr, axis=1)
    dot = jax.lax.dot_general(xr, xall, (((1,), (1,)), ((), ())),
                              preferred_element_type=f32, precision=PREC)
    d = sqr[:, None] + sqall[None, :] - 2.0 * dot

    a_ref[0] = jax.lax.dot_general(xr, Wd_ref[...], (((1,), (0,)), ((), ())),
                                   preferred_element_type=f32,
                                   precision=PREC) + ba_ref[...]
    y_ref[0] = jax.lax.dot_general(xr, Wn_ref[...], (((1,), (0,)), ((), ())),
                                   preferred_element_type=f32, precision=PREC)

    HALF = NPTS // 2
    iotap = jax.lax.broadcasted_iota(jnp.int32, (R, HALF), 1)
    dlow, dhigh = d[:, :HALF], d[:, HALF:]
    lo = jnp.minimum(dlow, dhigh)
    hi = jnp.maximum(dlow, dhigh)
    low_first = dlow <= dhigh
    for k in range(K):
        mj = jnp.min(lo, axis=1)
        cand = jnp.where(lo <= mj[:, None], iotap, HALF)
        j = jnp.min(cand, axis=1)
        ohp = iotap == j[:, None]
        sel_low = jnp.max(jnp.where(ohp & low_first, 1, 0), axis=1)
        idx_ref[0, k, :] = j + (1 - sel_low) * HALF + boff
        lo = jnp.where(ohp, hi, lo)
        hi = jnp.where(ohp, BIG, hi)
        low_first = low_first ^ ohp


def _knn(x, Wd, Wn, ba):
    C = x.shape[-1]
    H = Wd.shape[1]
    HG = Wn.shape[1]
    return pl.pallas_call(
        _knn_body,
        grid=(B, NPTS // R),
        in_specs=[
            pl.BlockSpec((1, NPTS, C), lambda b, r: (b, 0, 0)),
            pl.BlockSpec((1, R, C), lambda b, r: (b, r, 0)),
            pl.BlockSpec((C, H), lambda b, r: (0, 0)),
            pl.BlockSpec((C, HG), lambda b, r: (0, 0)),
            pl.BlockSpec((1, H), lambda b, r: (0, 0)),
        ],
        out_specs=[
            pl.BlockSpec((1, K, R), lambda b, r: (b, 0, r)),
            pl.BlockSpec((1, R, HG), lambda b, r: (b, r, 0)),
            pl.BlockSpec((1, R, H), lambda b, r: (b, r, 0)),
        ],
        out_shape=[
            jax.ShapeDtypeStruct((B, K, NPTS), jnp.int32),
            jax.ShapeDtypeStruct((B, NPTS, HG), jnp.float32),
            jax.ShapeDtypeStruct((B, NPTS, H), jnp.float32),
        ],
    )(x, x, Wd, Wn, ba)


def _sc_gather(idx_flat, table):
    H = table.shape[1]
    CH = 128
    NW = 32
    per_w = NIDX // NW
    mesh = plsc.VectorSubcoreMesh(core_axis_name="c", subcore_axis_name="s")

    @functools.partial(
        pl.kernel, mesh=mesh,
        out_type=jax.ShapeDtypeStruct((NIDX, H), jnp.float32),
        scratch_types=[
            pltpu.VMEM((CH,), jnp.int32),
            pltpu.VMEM((CH, H), jnp.float32),
            pltpu.SemaphoreType.DMA,
        ],
    )
    def gk(idx_hbm, table_hbm, out_hbm, idx_v, rows_v, sem):
        wid = lax.axis_index("s") * 2 + lax.axis_index("c")
        base = wid * per_w

        def body(t, carry):
            off = base + t * CH
            pltpu.sync_copy(idx_hbm.at[pl.ds(off, CH)], idx_v)
            pltpu.async_copy(table_hbm.at[idx_v], rows_v, sem).wait()
            pltpu.sync_copy(rows_v, out_hbm.at[pl.ds(off, CH)])
            return carry

        lax.fori_loop(0, per_w // CH, body, 0)

    return gk(idx_flat, table)


def _edge_mlp_body(g_ref, a_ref, Wb_ref, bb_ref, out_ref):
    f32 = jnp.float32
    PREC = jax.lax.Precision.HIGHEST
    a = a_ref[0]
    H = a.shape[-1]
    g = g_ref[0][:, :, :H]
    u = jnp.maximum(g + a[None, :, :], 0.0).reshape(K * R, H)
    h = jax.lax.dot_general(u, Wb_ref[...], (((1,), (0,)), ((), ())),
                            preferred_element_type=f32, precision=PREC)
    h = h.reshape(K, R, Wb_ref.shape[1])
    out_ref[0] = jnp.max(h, axis=0) + bb_ref[...]


def _edge_mlp(g, a, Wb, bb):
    HG = g.shape[-1]
    H = a.shape[-1]
    F = Wb.shape[1]
    return pl.pallas_call(
        _edge_mlp_body,
        grid=(B, NPTS // R),
        in_specs=[
            pl.BlockSpec((1, K, R, HG), lambda b, r: (b, 0, r, 0)),
            pl.BlockSpec((1, R, H), lambda b, r: (b, r, 0)),
            pl.BlockSpec((H, F), lambda b, r: (0, 0)),
            pl.BlockSpec((1, F), lambda b, r: (0, 0)),
        ],
        out_specs=pl.BlockSpec((1, R, F), lambda b, r: (b, r, 0)),
        out_shape=jax.ShapeDtypeStruct((B, NPTS, F), jnp.float32),
    )(g, a, Wb, bb)


def _edge_conv(x, Wd, Wn, ba, Wb, bb):
    H = Wd.shape[1]
    if H < 128:
        Wn = jnp.concatenate(
            [Wn, jnp.zeros((Wn.shape[0], 128 - H), jnp.float32)], axis=1)
    idx, y, a = _knn(x, Wd, Wn, ba)
    g = _sc_gather(idx.reshape(NIDX), y.reshape(B * NPTS, 128))
    return _edge_mlp(g.reshape(B, K, NPTS, 128), a, Wb, bb)


def _maxpool_body(x1_ref, x2_ref, x3_ref, x4_ref, out_ref):
    m1 = jnp.max(x1_ref[0], axis=0)
    m2 = jnp.max(x2_ref[0], axis=0)
    m3 = jnp.max(x3_ref[0], axis=0)
    m4 = jnp.max(x4_ref[0], axis=0)
    out_ref[...] = jnp.concatenate([m1, m2, m3, m4], axis=0)[None, None, :]


def _maxpool(x1, x2, x3, x4):
    return pl.pallas_call(
        _maxpool_body,
        grid=(B,),
        in_specs=[
            pl.BlockSpec((1, NPTS, 64), lambda b: (b, 0, 0)),
            pl.BlockSpec((1, NPTS, 64), lambda b: (b, 0, 0)),
            pl.BlockSpec((1, NPTS, 64), lambda b: (b, 0, 0)),
            pl.BlockSpec((1, NPTS, 128), lambda b: (b, 0, 0)),
        ],
        out_specs=pl.BlockSpec((1, 1, 320), lambda b: (b, 0, 0)),
        out_shape=jax.ShapeDtypeStruct((B, 1, 320), jnp.float32),
    )(x1, x2, x3, x4).reshape(B, 320)


def _decoder_body(pooled_ref, tooth_ref, emb_table_ref, cembT_ref, cemb_b_ref,
                  encWa_ref, encWb_ref, enc_b_ref, dW1_ref, db1_ref,
                  dW2_ref, db2_ref, out_ref):
    f32 = jnp.float32
    PREC = jax.lax.Precision.HIGHEST

    def mm(u, v):
        return jax.lax.dot_general(u, v, (((1,), (0,)), ((), ())),
                                   preferred_element_type=f32, precision=PREC)

    tooth = tooth_ref[...]
    oh = (jax.lax.broadcasted_iota(jnp.int32, (B, 33), 1)
          == tooth).astype(f32)
    emb = mm(oh, emb_table_ref[...])
    emb = mm(emb, cembT_ref[...]) + cemb_b_ref[...]
    h = mm(pooled_ref[...], encWa_ref[...]) + mm(emb, encWb_ref[...])
    h = jnp.maximum(h + enc_b_ref[...], 0.0)
    h = jnp.maximum(mm(h, dW1_ref[...]) + db1_ref[...], 0.0)
    out_ref[...] = mm(h, dW2_ref[...]) + db2_ref[...]


def _decoder(pooled, tooth_n, emb_table, conv_emb_W, conv_emb_b,
             enc_W, enc_b, dec_W1, dec_b1, dec_W2, dec_b2):
    return pl.pallas_call(
        _decoder_body,
        out_shape=jax.ShapeDtypeStruct((B, 3072), jnp.float32),
    )(pooled, tooth_n.reshape(B, 1), emb_table, conv_emb_W.T,
      conv_emb_b.reshape(1, 64), enc_W[:320], enc_W[320:],
      enc_b.reshape(1, 512), dec_W1, dec_b1.reshape(1, 1024),
      dec_W2, dec_b2.reshape(1, 3072))


def kernel(pos, batch, tooth_n, emb_table, conv_emb_W, conv_emb_b,
           W1a, b1a, W1b, b1b, W2a, b2a, W2b, b2b,
           W3a, b3a, W3b, b3b, W4a, b4a, W4b, b4b,
           enc_W, enc_b, dec_W1, dec_b1, dec_W2, dec_b2):
    x0 = pos.reshape(B, NPTS, 3)
    x0 = jnp.concatenate([x0, jnp.zeros((B, NPTS, 5), jnp.float32)], axis=-1)

    def split(Wa, cpad=None):
        C = Wa.shape[0] // 2
        top, bot = Wa[:C], Wa[C:]
        Wd, Wn = top - bot, bot
        if cpad is not None and cpad > C:
            z = jnp.zeros((cpad - C, Wa.shape[1]), jnp.float32)
            Wd = jnp.concatenate([Wd, z], axis=0)
            Wn = jnp.concatenate([Wn, z], axis=0)
        return Wd, Wn

    Wd1, Wn1 = split(W1a, 8)
    x1 = _edge_conv(x0, Wd1, Wn1, b1a.reshape(1, -1), W1b, b1b.reshape(1, -1))
    Wd2, Wn2 = split(W2a)
    x2 = _edge_conv(x1, Wd2, Wn2, b2a.reshape(1, -1), W2b, b2b.reshape(1, -1))
    Wd3, Wn3 = split(W3a)
    x3 = _edge_conv(x2, Wd3, Wn3, b3a.reshape(1, -1), W3b, b3b.reshape(1, -1))
    Wd4, Wn4 = split(W4a)
    x4 = _edge_conv(x3, Wd4, Wn4, b4a.reshape(1, -1), W4b, b4b.reshape(1, -1))

    pooled = _maxpool(x1, x2, x3, x4)
    out = _decoder(pooled, tooth_n, emb_table, conv_emb_W, conv_emb_b,
                   enc_W, enc_b, dec_W1, dec_b1, dec_W2, dec_b2)
    return out.reshape(B, 1024, 3)


# speedup vs baseline: 1.3676x; 1.1745x over previous
import functools
import jax
import jax.numpy as jnp
from jax import lax
from jax.experimental import pallas as pl
from jax.experimental.pallas import tpu as pltpu
from jax.experimental.pallas import tpu_sc as plsc

K = 20
B = 8
NPTS = 2048
R = 256
BIG = 3.0e38
NIDX = B * K * NPTS


def _knn_body(x_full_ref, x_row_ref, Wd_ref, Wn_ref, ba_ref,
              idx_ref, y_ref, a_ref):
    xall = x_full_ref[0]
    xr = x_row_ref[0]
    f32 = jnp.float32
    PREC = jax.lax.Precision.HIGHEST
    boff = pl.program_id(0) * NPTS

    sqall = jnp.sum(xall * xall, axis=1)
    sqr = jnp.sum(xr * xr, axis=1)
    dot = jax.lax.dot_general(xall, xr, (((1,), (1,)), ((), ())),
                              preferred_element_type=f32, precision=PREC)
    d = sqall[:, None] + sqr[None, :] - 2.0 * dot

    a_ref[0] = jax.lax.dot_general(xr, Wd_ref[...], (((1,), (0,)), ((), ())),
                                   preferred_element_type=f32,
                                   precision=PREC) + ba_ref[...]
    y_ref[0] = jax.lax.dot_general(xr, Wn_ref[...], (((1,), (0,)), ((), ())),
                                   preferred_element_type=f32, precision=PREC)

    HALF = NPTS // 2
    iotap = jax.lax.broadcasted_iota(jnp.int32, (HALF, R), 0)
    dlow, dhigh = d[:HALF, :], d[HALF:, :]
    lo = jnp.minimum(dlow, dhigh)
    hi = jnp.maximum(dlow, dhigh)
    lf = jnp.where(dlow <= dhigh, 1, 0)
    key0 = iotap * 2
    for k in range(K):
        mj = jnp.min(lo, axis=0)
        key = jnp.where(lo <= mj[None, :], key0 + (1 - lf), 2 * NPTS)
        j2 = jnp.min(key, axis=0)
        j = j2 // 2
        idx_ref[0, k, :] = j + (j2 - 2 * j) * HALF + boff
        ohp = iotap == j[None, :]
        lo = jnp.where(ohp, hi, lo)
        hi = jnp.where(ohp, BIG, hi)
        lf = jnp.where(ohp, 1 - lf, lf)


def _knn(x, Wd, Wn, ba):
    C = x.shape[-1]
    H = Wd.shape[1]
    HG = Wn.shape[1]
    return pl.pallas_call(
        _knn_body,
        grid=(B, NPTS // R),
        in_specs=[
            pl.BlockSpec((1, NPTS, C), lambda b, r: (b, 0, 0)),
            pl.BlockSpec((1, R, C), lambda b, r: (b, r, 0)),
            pl.BlockSpec((C, H), lambda b, r: (0, 0)),
            pl.BlockSpec((C, HG), lambda b, r: (0, 0)),
            pl.BlockSpec((1, H), lambda b, r: (0, 0)),
        ],
        out_specs=[
            pl.BlockSpec((1, K, R), lambda b, r: (b, 0, r)),
            pl.BlockSpec((1, R, HG), lambda b, r: (b, r, 0)),
            pl.BlockSpec((1, R, H), lambda b, r: (b, r, 0)),
        ],
        out_shape=[
            jax.ShapeDtypeStruct((B, K, NPTS), jnp.int32),
            jax.ShapeDtypeStruct((B, NPTS, HG), jnp.float32),
            jax.ShapeDtypeStruct((B, NPTS, H), jnp.float32),
        ],
    )(x, x, Wd, Wn, ba)


def _sc_gather(idx_flat, table):
    H = table.shape[1]
    CH = 128
    NW = 32
    per_w = NIDX // NW
    mesh = plsc.VectorSubcoreMesh(core_axis_name="c", subcore_axis_name="s")

    @functools.partial(
        pl.kernel, mesh=mesh,
        out_type=jax.ShapeDtypeStruct((NIDX, H), jnp.float32),
        scratch_types=[
            pltpu.VMEM((CH,), jnp.int32),
            pltpu.VMEM((CH, H), jnp.float32),
            pltpu.SemaphoreType.DMA,
        ],
    )
    def gk(idx_hbm, table_hbm, out_hbm, idx_v, rows_v, sem):
        wid = lax.axis_index("s") * 2 + lax.axis_index("c")
        base = wid * per_w

        def body(t, carry):
            off = base + t * CH
            pltpu.sync_copy(idx_hbm.at[pl.ds(off, CH)], idx_v)
            pltpu.async_copy(table_hbm.at[idx_v], rows_v, sem).wait()
            pltpu.sync_copy(rows_v, out_hbm.at[pl.ds(off, CH)])
            return carry

        lax.fori_loop(0, per_w // CH, body, 0)

    return gk(idx_flat, table)


def _edge_mlp_body(g_ref, a_ref, Wb_ref, bb_ref, out_ref):
    f32 = jnp.float32
    PREC = jax.lax.Precision.HIGHEST
    a = a_ref[0]
    H = a.shape[-1]
    g = g_ref[0][:, :, :H]
    u = jnp.maximum(g + a[None, :, :], 0.0).reshape(K * R, H)
    h = jax.lax.dot_general(u, Wb_ref[...], (((1,), (0,)), ((), ())),
                            preferred_element_type=f32, precision=PREC)
    h = h.reshape(K, R, Wb_ref.shape[1])
    out_ref[0] = jnp.max(h, axis=0) + bb_ref[...]


def _edge_mlp(g, a, Wb, bb):
    HG = g.shape[-1]
    H = a.shape[-1]
    F = Wb.shape[1]
    return pl.pallas_call(
        _edge_mlp_body,
        grid=(B, NPTS // R),
        in_specs=[
            pl.BlockSpec((1, K, R, HG), lambda b, r: (b, 0, r, 0)),
            pl.BlockSpec((1, R, H), lambda b, r: (b, r, 0)),
            pl.BlockSpec((H, F), lambda b, r: (0, 0)),
            pl.BlockSpec((1, F), lambda b, r: (0, 0)),
        ],
        out_specs=pl.BlockSpec((1, R, F), lambda b, r: (b, r, 0)),
        out_shape=jax.ShapeDtypeStruct((B, NPTS, F), jnp.float32),
    )(g, a, Wb, bb)


def _edge_conv(x, Wd, Wn, ba, Wb, bb):
    H = Wd.shape[1]
    if H < 128:
        Wn = jnp.concatenate(
            [Wn, jnp.zeros((Wn.shape[0], 128 - H), jnp.float32)], axis=1)
    idx, y, a = _knn(x, Wd, Wn, ba)
    g = _sc_gather(idx.reshape(NIDX), y.reshape(B * NPTS, 128))
    return _edge_mlp(g.reshape(B, K, NPTS, 128), a, Wb, bb)


def _maxpool_body(x1_ref, x2_ref, x3_ref, x4_ref, out_ref):
    m1 = jnp.max(x1_ref[0], axis=0)
    m2 = jnp.max(x2_ref[0], axis=0)
    m3 = jnp.max(x3_ref[0], axis=0)
    m4 = jnp.max(x4_ref[0], axis=0)
    out_ref[...] = jnp.concatenate([m1, m2, m3, m4], axis=0)[None, None, :]


def _maxpool(x1, x2, x3, x4):
    return pl.pallas_call(
        _maxpool_body,
        grid=(B,),
        in_specs=[
            pl.BlockSpec((1, NPTS, 64), lambda b: (b, 0, 0)),
            pl.BlockSpec((1, NPTS, 64), lambda b: (b, 0, 0)),
            pl.BlockSpec((1, NPTS, 64), lambda b: (b, 0, 0)),
            pl.BlockSpec((1, NPTS, 128), lambda b: (b, 0, 0)),
        ],
        out_specs=pl.BlockSpec((1, 1, 320), lambda b: (b, 0, 0)),
        out_shape=jax.ShapeDtypeStruct((B, 1, 320), jnp.float32),
    )(x1, x2, x3, x4).reshape(B, 320)


def _decoder_body(pooled_ref, tooth_ref, emb_table_ref, cembT_ref, cemb_b_ref,
                  encWa_ref, encWb_ref, enc_b_ref, dW1_ref, db1_ref,
                  dW2_ref, db2_ref, out_ref):
    f32 = jnp.float32
    PREC = jax.lax.Precision.HIGHEST

    def mm(u, v):
        return jax.lax.dot_general(u, v, (((1,), (0,)), ((), ())),
                                   preferred_element_type=f32, precision=PREC)

    tooth = tooth_ref[...]
    oh = (jax.lax.broadcasted_iota(jnp.int32, (B, 33), 1)
          == tooth).astype(f32)
    emb = mm(oh, emb_table_ref[...])
    emb = mm(emb, cembT_ref[...]) + cemb_b_ref[...]
    h = mm(pooled_ref[...], encWa_ref[...]) + mm(emb, encWb_ref[...])
    h = jnp.maximum(h + enc_b_ref[...], 0.0)
    h = jnp.maximum(mm(h, dW1_ref[...]) + db1_ref[...], 0.0)
    out_ref[...] = mm(h, dW2_ref[...]) + db2_ref[...]


def _decoder(pooled, tooth_n, emb_table, conv_emb_W, conv_emb_b,
             enc_W, enc_b, dec_W1, dec_b1, dec_W2, dec_b2):
    return pl.pallas_call(
        _decoder_body,
        out_shape=jax.ShapeDtypeStruct((B, 3072), jnp.float32),
    )(pooled, tooth_n.reshape(B, 1), emb_table, conv_emb_W.T,
      conv_emb_b.reshape(1, 64), enc_W[:320], enc_W[320:],
      enc_b.reshape(1, 512), dec_W1, dec_b1.reshape(1, 1024),
      dec_W2, dec_b2.reshape(1, 3072))


def kernel(pos, batch, tooth_n, emb_table, conv_emb_W, conv_emb_b,
           W1a, b1a, W1b, b1b, W2a, b2a, W2b, b2b,
           W3a, b3a, W3b, b3b, W4a, b4a, W4b, b4b,
           enc_W, enc_b, dec_W1, dec_b1, dec_W2, dec_b2):
    x0 = pos.reshape(B, NPTS, 3)
    x0 = jnp.concatenate([x0, jnp.zeros((B, NPTS, 5), jnp.float32)], axis=-1)

    def split(Wa, cpad=None):
        C = Wa.shape[0] // 2
        top, bot = Wa[:C], Wa[C:]
        Wd, Wn = top - bot, bot
        if cpad is not None and cpad > C:
            z = jnp.zeros((cpad - C, Wa.shape[1]), jnp.float32)
            Wd = jnp.concatenate([Wd, z], axis=0)
            Wn = jnp.concatenate([Wn, z], axis=0)
        return Wd, Wn

    Wd1, Wn1 = split(W1a, 8)
    x1 = _edge_conv(x0, Wd1, Wn1, b1a.reshape(1, -1), W1b, b1b.reshape(1, -1))
    Wd2, Wn2 = split(W2a)
    x2 = _edge_conv(x1, Wd2, Wn2, b2a.reshape(1, -1), W2b, b2b.reshape(1, -1))
    Wd3, Wn3 = split(W3a)
    x3 = _edge_conv(x2, Wd3, Wn3, b3a.reshape(1, -1), W3b, b3b.reshape(1, -1))
    Wd4, Wn4 = split(W4a)
    x4 = _edge_conv(x3, Wd4, Wn4, b4a.reshape(1, -1), W4b, b4b.reshape(1, -1))

    pooled = _maxpool(x1, x2, x3, x4)
    out = _decoder(pooled, tooth_n, emb_table, conv_emb_W, conv_emb_b,
                   enc_W, enc_b, dec_W1, dec_b1, dec_W2, dec_b2)
    return out.reshape(B, 1024, 3)
